# Initial kernel scaffold; baseline (speedup 1.0000x reference)
import jax
import jax.numpy as jnp
from jax.experimental import pallas as pl


def kernel(x, graph, params):
    raise NotImplementedError("write your pallas kernel here")


# speedup vs baseline: 2.2019x; 2.2019x over previous
import functools

import jax
import jax.numpy as jnp
from jax import lax
from jax.experimental import pallas as pl
from jax.experimental.pallas import tpu as pltpu
from jax.experimental.pallas import tpu_sc as plsc

_NC = 2
_NS = 16
_NW = _NC * _NS


def _sc_mesh():
    return plsc.VectorSubcoreMesh(core_axis_name="c", subcore_axis_name="s")


def _sc_gather(table, idx, chunk):
    B = idx.shape[0]
    D = table.shape[1]
    per_w = B // _NW
    n_chunks = per_w // chunk

    kfn = functools.partial(
        pl.kernel,
        out_type=jax.ShapeDtypeStruct((B, D), table.dtype),
        mesh=_sc_mesh(),
        scratch_types=[
            pltpu.VMEM((chunk,), jnp.int32),
            pltpu.VMEM((chunk, D), table.dtype),
            pltpu.SemaphoreType.DMA,
        ],
        compiler_params=pltpu.CompilerParams(use_tc_tiling_on_sc=False),
    )

    @kfn
    def run(table_hbm, idx_hbm, out_hbm, idx_v, rows_v, sem):
        wid = lax.axis_index("s") * _NC + lax.axis_index("c")
        base = wid * per_w

        def body(j, carry):
            o = base + j * chunk
            pltpu.sync_copy(idx_hbm.at[pl.ds(o, chunk)], idx_v)
            pltpu.async_copy(table_hbm.at[idx_v], rows_v, sem).wait()
            pltpu.sync_copy(rows_v, out_hbm.at[pl.ds(o, chunk)])
            return carry

        lax.fori_loop(0, n_chunks, body, 0)

    return run(table, idx)


def _sc_segment_sum(vals, idx2, acc_rows, chunk, zeros_hbm):
    E, D = vals.shape
    per_tile = E // _NS
    n_chunks = per_tile // chunk
    rows_per_tile = acc_rows // _NS

    kfn = functools.partial(
        pl.kernel,
        out_type=jax.ShapeDtypeStruct((_NC, acc_rows, D), jnp.float32),
        mesh=_sc_mesh(),
        scratch_types=[
            pltpu.VMEM((chunk,), jnp.int32),
            pltpu.VMEM((chunk, D), jnp.float32),
            pltpu.VMEM_SHARED((acc_rows, D), jnp.float32),
        ],
        compiler_params=pltpu.CompilerParams(use_tc_tiling_on_sc=False),
    )

    @kfn
    def run(v_hbm, i_hbm, z_hbm, out_hbm, idx_v, rows_v, acc_sh):
        cid = lax.axis_index("c")
        sid = lax.axis_index("s")
        r0 = sid * rows_per_tile

        pltpu.sync_copy(z_hbm.at[pl.ds(r0, rows_per_tile)],
                        acc_sh.at[pl.ds(r0, rows_per_tile)])
        plsc.subcore_barrier()

        def body(j, carry):
            o = sid * per_tile + j * chunk
            pltpu.sync_copy(i_hbm.at[cid, pl.ds(o, chunk)], idx_v)
            pltpu.sync_copy(v_hbm.at[pl.ds(o, chunk)], rows_v)
            pltpu.sync_copy(rows_v, acc_sh.at[idx_v], add=True)
            return carry

        lax.fori_loop(0, n_chunks, body, 0)
        plsc.subcore_barrier()
        pltpu.sync_copy(acc_sh.at[pl.ds(r0, rows_per_tile)],
                        out_hbm.at[cid, pl.ds(r0, rows_per_tile)])

    return run(vals, idx2, zeros_hbm)


def _ln_silu(y, gm, bt):
    mu = jnp.mean(y, axis=-1, keepdims=True)
    d = y - mu
    var = jnp.mean(d * d, axis=-1, keepdims=True)
    y = d * lax.rsqrt(var + 1e-5) * gm + bt
    return y * jax.nn.sigmoid(y)


def _tc_mlp(parts, w1, vec1, w2, vec2, *, out_norm, residual=None,
            l2norm=False, block=1000):
    R = parts[0][0].shape[0]
    dims = [g[0].shape[1] for g in parts]
    H = w1.shape[1]
    O = w2.shape[1]
    grid = R // block
    xs = [a for g in parts for a in g]
    sizes = [len(g) for g in parts]
    has_res = residual is not None

    def body(*refs):
        k = len(xs)
        xrefs = refs[:k]
        res_ref = None
        if has_res:
            res_ref = refs[k]
            k += 1
        w1r, v1r, w2r, v2r, outr = refs[k:k + 5]

        acc = None
        off = 0
        xi = 0
        for gsize, Di in zip(sizes, dims):
            xv = xrefs[xi][...].astype(jnp.float32)
            for t in range(1, gsize):
                xv = xv + xrefs[xi + t][...].astype(jnp.float32)
            xi += gsize
            term = jnp.dot(xv, w1r[off:off + Di, :],
                           preferred_element_type=jnp.float32)
            acc = term if acc is None else acc + term
            off += Di
        acc = acc + v1r[0:1, :]
        acc = _ln_silu(acc, v1r[1:2, :], v1r[2:3, :])
        y = jnp.dot(acc, w2r[...], preferred_element_type=jnp.float32)
        y = y + v2r[0:1, :]
        if out_norm:
            y = _ln_silu(y, v2r[1:2, :], v2r[2:3, :])
        if has_res:
            y = y + res_ref[...].astype(jnp.float32)
        if l2norm:
            nrm = jnp.sqrt(jnp.sum(y * y, axis=-1, keepdims=True))
            y = y / jnp.maximum(nrm, 1e-12)
        outr[...] = y

    in_arrays = list(xs)
    in_specs = [pl.BlockSpec((block, a.shape[1]), lambda i: (i, 0)) for a in xs]
    if has_res:
        in_arrays.append(residual)
        in_specs.append(pl.BlockSpec((block, O), lambda i: (i, 0)))
    for warr in (w1, vec1, w2, vec2):
        in_arrays.append(warr)
        in_specs.append(pl.BlockSpec(warr.shape, lambda i: (0, 0)))

    return pl.pallas_call(
        body,
        grid=(grid,),
        in_specs=in_specs,
        out_specs=pl.BlockSpec((block, O), lambda i: (i, 0)),
        out_shape=jax.ShapeDtypeStruct((R, O), jnp.float32),
    )(*in_arrays)


def _pack_vecs(layer, width):
    b = layer["b"]
    g = layer.get("g", jnp.zeros((width,), jnp.float32))
    be = layer.get("be", jnp.zeros((width,), jnp.float32))
    return jnp.stack([b, g, be], axis=0)


def kernel(x, graph, params):
    N = x.shape[0]
    E = graph.shape[1]
    src = graph[0]
    dst = graph[1]

    idx_int = jnp.stack([src, dst], axis=1).reshape(2 * E)

    ne = params["node_encoder"]
    ee = params["edge_encoder"]
    out_p = params["output"]

    x_pad = jnp.pad(x, ((0, 0), (0, 5)))
    w1n = jnp.zeros((8, ne[0]["W"].shape[1]), jnp.float32).at[0:3].set(ne[0]["W"])

    H = ee[0]["W"].shape[1]
    w1e = (jnp.zeros((16, H), jnp.float32)
           .at[0:3].set(ee[0]["W"][0:3])
           .at[8:11].set(ee[0]["W"][3:6]))

    gx = _sc_gather(x_pad, idx_int, chunk=1000).reshape(E, 16)

    nodes = _tc_mlp([[x_pad]], w1n, _pack_vecs(ne[0], H),
                    ne[1]["W"], _pack_vecs(ne[1], ne[1]["W"].shape[1]),
                    out_norm=True)
    edges = _tc_mlp([[gx]], w1e, _pack_vecs(ee[0], H),
                    ee[1]["W"], _pack_vecs(ee[1], ee[1]["W"].shape[1]),
                    out_norm=True)

    half = N // 2
    n_trash = 512
    acc_rows = ((half + n_trash + _NS - 1) // _NS) * _NS
    trash = half + (jnp.arange(E, dtype=jnp.int32) % n_trash)
    dst0 = jnp.where(dst < half, dst, trash)
    dst1 = jnp.where(dst >= half, dst - half, trash)
    idx2 = jnp.stack([dst0, dst1])
    zeros_seg = jnp.zeros((acc_rows, edges.shape[1]), jnp.float32)

    for cp in params["cells"]:
        nw = cp["node"]
        ew = cp["edge"]
        parts = _sc_segment_sum(edges, idx2, acc_rows, 1000, zeros_seg)
        msg = jnp.concatenate([parts[0, :half], parts[1, :half]], axis=0)
        nodes = _tc_mlp([[nodes], [msg]],
                        nw[0]["W"], _pack_vecs(nw[0], nw[0]["W"].shape[1]),
                        nw[1]["W"], _pack_vecs(nw[1], nw[1]["W"].shape[1]),
                        out_norm=True, residual=nodes)
        g = _sc_gather(nodes, idx_int, chunk=1000).reshape(E, 2 * nodes.shape[1])
        edges = _tc_mlp([[g], [edges]],
                        ew[0]["W"], _pack_vecs(ew[0], ew[0]["W"].shape[1]),
                        ew[1]["W"], _pack_vecs(ew[1], ew[1]["W"].shape[1]),
                        out_norm=True, residual=edges)

    emb = _tc_mlp([[nodes]], out_p[0]["W"],
                  _pack_vecs(out_p[0], out_p[0]["W"].shape[1]),
                  out_p[1]["W"], _pack_vecs(out_p[1], out_p[1]["W"].shape[1]),
                  out_norm=False, l2norm=True)
    return emb, nodes, edges


# speedup vs baseline: 3.8319x; 1.7403x over previous
import functools

import jax
import jax.numpy as jnp
from jax import lax
from jax.experimental import pallas as pl
from jax.experimental.pallas import tpu as pltpu
from jax.experimental.pallas import tpu_sc as plsc

_NC = 2
_NS = 16
_NW = _NC * _NS


def _sc_mesh():
    return plsc.VectorSubcoreMesh(core_axis_name="c", subcore_axis_name="s")


def _sc_gather(table, idx, chunk):
    B = idx.shape[0]
    D = table.shape[1]
    per_w = B // _NW
    n_chunks = per_w // chunk

    kfn = functools.partial(
        pl.kernel,
        out_type=jax.ShapeDtypeStruct((B, D), table.dtype),
        mesh=_sc_mesh(),
        scratch_types=[
            pltpu.VMEM((chunk,), jnp.int32),
            pltpu.VMEM((chunk, D), table.dtype),
            pltpu.SemaphoreType.DMA,
        ],
        compiler_params=pltpu.CompilerParams(use_tc_tiling_on_sc=False),
    )

    @kfn
    def run(table_hbm, idx_hbm, out_hbm, idx_v, rows_v, sem):
        wid = lax.axis_index("s") * _NC + lax.axis_index("c")
        base = wid * per_w

        def body(j, carry):
            o = base + j * chunk
            pltpu.sync_copy(idx_hbm.at[pl.ds(o, chunk)], idx_v)
            pltpu.async_copy(table_hbm.at[idx_v], rows_v, sem).wait()
            pltpu.sync_copy(rows_v, out_hbm.at[pl.ds(o, chunk)])
            return carry

        lax.fori_loop(0, n_chunks, body, 0)

    return run(table, idx)


def _sc_segment_sum(vals, idx2, acc_rows, chunk, zeros_hbm):
    E, D = vals.shape
    per_tile = E // _NS
    n_chunks = per_tile // chunk
    rows_per_tile = acc_rows // _NS

    kfn = functools.partial(
        pl.kernel,
        out_type=jax.ShapeDtypeStruct((_NC, acc_rows, D), jnp.float32),
        mesh=_sc_mesh(),
        scratch_types=[
            pltpu.VMEM((chunk,), jnp.int32),
            pltpu.VMEM((chunk, D), jnp.float32),
            pltpu.VMEM_SHARED((acc_rows, D), jnp.float32),
        ],
        compiler_params=pltpu.CompilerParams(use_tc_tiling_on_sc=False),
    )

    @kfn
    def run(v_hbm, i_hbm, z_hbm, out_hbm, idx_v, rows_v, acc_sh):
        cid = lax.axis_index("c")
        sid = lax.axis_index("s")
        r0 = sid * rows_per_tile

        pltpu.sync_copy(z_hbm.at[pl.ds(r0, rows_per_tile)],
                        acc_sh.at[pl.ds(r0, rows_per_tile)])
        plsc.subcore_barrier()

        def body(j, carry):
            o = sid * per_tile + j * chunk
            pltpu.sync_copy(i_hbm.at[cid, pl.ds(o, chunk)], idx_v)
            pltpu.sync_copy(v_hbm.at[pl.ds(o, chunk)], rows_v)
            pltpu.sync_copy(rows_v, acc_sh.at[idx_v], add=True)
            return carry

        lax.fori_loop(0, n_chunks, body, 0)
        plsc.subcore_barrier()
        pltpu.sync_copy(acc_sh.at[pl.ds(r0, rows_per_tile)],
                        out_hbm.at[cid, pl.ds(r0, rows_per_tile)])

    return run(vals, idx2, zeros_hbm)


def _group_ind(width, gsize):
    g = width // gsize
    r = lax.broadcasted_iota(jnp.int32, (width, g), 0) // gsize
    c = lax.broadcasted_iota(jnp.int32, (width, g), 1)
    return (r == c).astype(jnp.float32)


def _ln_silu_g(y, gm, bt, gsize):
    a = _group_ind(y.shape[1], gsize)
    mu = jnp.dot(jnp.dot(y, a, preferred_element_type=jnp.float32) / gsize,
                 a.T, preferred_element_type=jnp.float32)
    c = y - mu
    var = jnp.dot(jnp.dot(c * c, a, preferred_element_type=jnp.float32) / gsize,
                  a.T, preferred_element_type=jnp.float32)
    y = c * lax.rsqrt(var + 1e-5) * gm + bt
    return y * jax.nn.sigmoid(y)


def _tc_mlp_packed(xs, specs, w1s, vec1, w2, vec2, *, grid, g1, g2,
                   out_shape, out_block, out_norm, res_idx=None, l2g=None):
    n = len(xs)

    def body(*refs):
        xr = refs[:n]
        w1r = refs[n:2 * n]
        v1r, w2r, v2r, outr = refs[2 * n:2 * n + 4]

        acc = None
        for i in range(n):
            t = jnp.dot(xr[i][...], w1r[i][...],
                        preferred_element_type=jnp.float32)
            acc = t if acc is None else acc + t
        acc = acc + v1r[0:1, :]
        acc = _ln_silu_g(acc, v1r[1:2, :], v1r[2:3, :], g1)
        y = jnp.dot(acc, w2r[...], preferred_element_type=jnp.float32)
        y = y + v2r[0:1, :]
        if out_norm:
            y = _ln_silu_g(y, v2r[1:2, :], v2r[2:3, :], g2)
        if res_idx is not None:
            y = y + xr[res_idx][...]
        if l2g is not None:
            a = _group_ind(y.shape[1], l2g)
            ss = jnp.dot(jnp.dot(y * y, a, preferred_element_type=jnp.float32),
                         a.T, preferred_element_type=jnp.float32)
            y = y / jnp.maximum(jnp.sqrt(ss), 1e-12)
        outr[...] = y

    in_arrays = list(xs) + list(w1s) + [vec1, w2, vec2]
    in_specs = [pl.BlockSpec(b, m) for (b, m) in specs]
    for warr in list(w1s) + [vec1, w2, vec2]:
        in_specs.append(pl.BlockSpec(warr.shape, lambda i: (0, 0)))

    return pl.pallas_call(
        body,
        grid=(grid,),
        in_specs=in_specs,
        out_specs=pl.BlockSpec(out_block, lambda i: (i, 0)),
        out_shape=jax.ShapeDtypeStruct(out_shape, jnp.float32),
    )(*in_arrays)


def _bd(m, k):
    di, do = m.shape
    out = jnp.zeros((di * k, do * k), m.dtype)
    for i in range(k):
        out = out.at[i * di:(i + 1) * di, i * do:(i + 1) * do].set(m)
    return out


def _vecs(layer, k):
    width = layer["b"].shape[0]
    b = jnp.tile(layer["b"], k)
    g = jnp.tile(layer.get("g", jnp.zeros((width,), jnp.float32)), k)
    be = jnp.tile(layer.get("be", jnp.zeros((width,), jnp.float32)), k)
    return jnp.stack([b, g, be], axis=0)


def kernel(x, graph, params):
    N = x.shape[0]
    E = graph.shape[1]
    L = 32
    src = graph[0]
    dst = graph[1]

    ne = params["node_encoder"]
    ee = params["edge_encoder"]
    out_p = params["output"]

    idx_int = jnp.stack([src, dst], axis=1).reshape(2 * E)

    half = N // 2
    acc_rows = ((half + 512 + 999) // 1000) * 1000
    trash = half + (jnp.arange(E, dtype=jnp.int32) % 512)
    idx2 = jnp.stack([jnp.where(dst < half, dst, trash),
                      jnp.where(dst >= half, dst - half, trash)])
    zeros_seg = jnp.zeros((acc_rows, L), jnp.float32)

    x_pack = jnp.pad(x, ((0, 0), (0, 5))).reshape(N * 8 // 128, 128)
    nx = x_pack.shape[0]
    w1n = jnp.zeros((8, 64), jnp.float32).at[0:3].set(ne[0]["W"])
    row_map = lambda i: (i, 0)
    nodes_p = _tc_mlp_packed(
        [x_pack], [((nx, 128), row_map)],
        [_bd(w1n, 16)], _vecs(ne[0], 16), _bd(ne[1]["W"], 16), _vecs(ne[1], 16),
        grid=1, g1=64, g2=32,
        out_shape=(nx, 512), out_block=(nx, 512), out_norm=True,
    ).reshape(N * L // 128, 128)

    gx = _sc_gather(x_pack.reshape(N, 8), idx_int, 1000)
    gxv = gx.reshape(2 * E * 8 // 128, 128)
    w1e = (jnp.zeros((16, 64), jnp.float32)
           .at[0:3].set(ee[0]["W"][0:3])
           .at[8:11].set(ee[0]["W"][3:6]))
    edges_p = _tc_mlp_packed(
        [gxv], [((1000, 128), row_map)],
        [_bd(w1e, 8)], _vecs(ee[0], 8), _bd(ee[1]["W"], 8), _vecs(ee[1], 8),
        grid=gxv.shape[0] // 1000, g1=64, g2=32,
        out_shape=(gxv.shape[0], 256), out_block=(1000, 256), out_norm=True,
    ).reshape(E * L // 128, 128)

    ep4 = E * L // 128
    np4 = N * L // 128
    for cp in params["cells"]:
        nw = cp["node"]
        ew = cp["edge"]
        parts = _sc_segment_sum(edges_p.reshape(E, L), idx2, acc_rows,
                                1000, zeros_seg)
        msg_p = jnp.concatenate([parts[0, :half], parts[1, :half]],
                                axis=0).reshape(np4, 128)
        nodes_p = _tc_mlp_packed(
            [nodes_p, msg_p],
            [((np4, 128), row_map), ((np4, 128), row_map)],
            [_bd(nw[0]["W"][0:L], 4), _bd(nw[0]["W"][L:2 * L], 4)],
            _vecs(nw[0], 4), _bd(nw[1]["W"], 4), _vecs(nw[1], 4),
            grid=1, g1=64, g2=32,
            out_shape=(np4, 128), out_block=(np4, 128), out_norm=True,
            res_idx=0,
        )
        g = _sc_gather(nodes_p.reshape(N, L), idx_int, 1000)
        xg = g.reshape(E * 2 * L // 256, 256)
        edges_p = _tc_mlp_packed(
            [xg, edges_p],
            [((1000, 256), row_map), ((1000, 128), row_map)],
            [_bd(ew[0]["W"][0:2 * L], 4), _bd(ew[0]["W"][2 * L:3 * L], 4)],
            _vecs(ew[0], 4), _bd(ew[1]["W"], 4), _vecs(ew[1], 4),
            grid=ep4 // 1000, g1=64, g2=32,
            out_shape=(ep4, 128), out_block=(1000, 128), out_norm=True,
            res_idx=1,
        )

    emb48 = _tc_mlp_packed(
        [nodes_p], [((np4, 128), row_map)],
        [_bd(out_p[0]["W"], 4)], _vecs(out_p[0], 4),
        _bd(out_p[1]["W"], 4), _vecs(out_p[1], 4),
        grid=1, g1=64, g2=12,
        out_shape=(np4, 48), out_block=(np4, 48), out_norm=False, l2g=12,
    )
    emb = emb48.reshape(N, 12)
    return emb, nodes_p.reshape(N, L), edges_p.reshape(E, L)
